# baseline (speedup 1.0000x reference)
import functools

import jax
import jax.numpy as jnp
from jax import lax
from jax.experimental import pallas as pl
from jax.experimental.pallas import tpu as pltpu
from jax.experimental.pallas import tpu_sc as plsc

_B, _TB, _H, _W = 16, 512, 64, 64
_C, _K = 64, 4
_NW = 32
_SLABS = 64
_THRESH = 2.0


def _compute_slab(slab, tabv, out_v, quarter):
    rtabs = [tabv[r, :] for r in range(16)]
    ctabs = [tabv[16 + r, :] for r in range(16)]

    @plsc.parallel_loop(0, 32, unroll=4)
    def pbody(p):
        rowb = (p >> 3) * 32 + (p & 7)
        for j in range(4):
            acc = None
            for k in range(_K):
                ir = rtabs[4 * j + k] + rowb
                g = plsc.load_gather(slab, [ir, ctabs[4 * j + k]])
                acc = g if k == 0 else acc + g
            y = jnp.where(acc >= _THRESH, jnp.float32(1.0), jnp.float32(0.0))
            out_v[quarter * 32 + p, pl.ds(16 * j, 16)] = y


def _sc_body(x_hbm, tab_hbm, out_hbm,
             tabv, slab_a, slab_b, slab_c, slab_d, ov_a, ov_b,
             gs_a, gs_b, gs_c, gs_d, os_a, os_b):
    wid = lax.axis_index("s") * 2 + lax.axis_index("c")
    b = wid // 2
    qbase = (wid % 2) * 16
    pltpu.sync_copy(tab_hbm, tabv)

    slabs = [slab_a, slab_b, slab_c, slab_d]
    outs = [ov_a, ov_b]
    gsems = [gs_a, gs_b, gs_c, gs_d]
    osems = [os_a, os_b]

    g0 = wid * _SLABS
    for i in range(4):
        pltpu.async_copy(x_hbm.at[g0 + i], slabs[i], gsems[i])

    def outer(it, _):
        base = it * 8
        for j in range(8):
            p = j & 3
            sdyn = base + j
            pltpu.make_async_copy(x_hbm.at[g0], slabs[p], gsems[p]).wait()
            op = (j // 4) & 1
            if j % 4 == 0:
                @pl.when(it > 0)
                def _():
                    pltpu.make_async_copy(
                        outs[op], out_hbm.at[b, qbase], osems[op]).wait()
            _compute_slab(slabs[p], tabv, outs[op], j % 4)
            @pl.when(sdyn + 4 < _SLABS)
            def _():
                pltpu.async_copy(
                    x_hbm.at[g0 + sdyn + 4], slabs[p], gsems[p])
            if j % 4 == 3:
                pltpu.async_copy(
                    outs[op], out_hbm.at[b, qbase + it * 2 + op],
                    osems[op])
        return 0

    lax.fori_loop(0, _SLABS // 8, outer, 0)
    pltpu.make_async_copy(outs[0], out_hbm.at[b, qbase], osems[0]).wait()
    pltpu.make_async_copy(outs[1], out_hbm.at[b, qbase], osems[1]).wait()


_sc_kernel = functools.partial(
    pl.kernel,
    out_type=jax.ShapeDtypeStruct((_B, 32, 128, _C), jnp.float32),
    mesh=plsc.VectorSubcoreMesh(core_axis_name="c", subcore_axis_name="s"),
    scratch_types=[
        pltpu.VMEM((32, 16), jnp.int32),
        pltpu.VMEM((128, 128), jnp.float32),
        pltpu.VMEM((128, 128), jnp.float32),
        pltpu.VMEM((128, 128), jnp.float32),
        pltpu.VMEM((128, 128), jnp.float32),
        pltpu.VMEM((128, _C), jnp.float32),
        pltpu.VMEM((128, _C), jnp.float32),
        pltpu.SemaphoreType.DMA,
        pltpu.SemaphoreType.DMA,
        pltpu.SemaphoreType.DMA,
        pltpu.SemaphoreType.DMA,
        pltpu.SemaphoreType.DMA,
        pltpu.SemaphoreType.DMA,
    ],
    compiler_params=pltpu.CompilerParams(needs_layout_passes=False),
)(_sc_body)


def kernel(x, rp_map_idx):
    t = jnp.transpose(x, (0, 2, 3, 1))
    t = t.reshape(_B, _H, 8, 8, 4, 128)
    t = jnp.transpose(t, (0, 1, 2, 4, 3, 5))
    x3 = t.reshape(2 * _B * _H, 128, 128)

    rp = rp_map_idx.astype(jnp.int32)
    rowoff = (rp // 128) * 8
    coloff = rp % 128
    rtab = jnp.transpose(rowoff.reshape(4, 16, _K), (0, 2, 1)).reshape(16, 16)
    ctab = jnp.transpose(coloff.reshape(4, 16, _K), (0, 2, 1)).reshape(16, 16)
    tab = jnp.concatenate([rtab, ctab], axis=0)

    y6 = _sc_kernel(x3, tab)
    y = y6.reshape(_B, 32, 2, _W, _C)
    y = jnp.transpose(y, (0, 4, 1, 2, 3)).reshape(_B, _C, _H, _W)
    return jnp.where(y >= 0.5, jnp.float32(1.0), jnp.float32(0.0))


# speedup vs baseline: 1.0385x; 1.0385x over previous
import functools

import jax
import jax.numpy as jnp
from jax import lax
from jax.experimental import pallas as pl
from jax.experimental.pallas import tpu as pltpu
from jax.experimental.pallas import tpu_sc as plsc

_B, _TB, _H, _W = 16, 512, 64, 64
_C, _K = 64, 4
_NW = 32
_SLABS = 64
_THRESH = 2.0


def _compute_slab(slab, tabv, out_v, quarter):
    rtabs = [tabv[r, :] for r in range(16)]
    ctabs = [tabv[16 + r, :] for r in range(16)]

    @plsc.parallel_loop(0, 32, unroll=8)
    def pbody(p):
        rowb = (p >> 3) * 32 + (p & 7)
        for j in range(4):
            acc = None
            for k in range(_K):
                ir = rtabs[4 * j + k] + rowb
                g = plsc.load_gather(slab, [ir, ctabs[4 * j + k]])
                acc = g if k == 0 else acc + g
            y = jnp.where(acc >= _THRESH, jnp.float32(1.0), jnp.float32(0.0))
            out_v[quarter * 32 + p, pl.ds(16 * j, 16)] = y


def _sc_body(x_hbm, tab_hbm, out_hbm,
             tabv, slab_a, slab_b, slab_c, slab_d, ov_a, ov_b,
             gs_a, gs_b, gs_c, gs_d, os_a, os_b):
    wid = lax.axis_index("s") * 2 + lax.axis_index("c")
    b = wid // 2
    qbase = (wid % 2) * 16
    pltpu.sync_copy(tab_hbm, tabv)

    slabs = [slab_a, slab_b, slab_c, slab_d]
    outs = [ov_a, ov_b]
    gsems = [gs_a, gs_b, gs_c, gs_d]
    osems = [os_a, os_b]

    g0 = wid * _SLABS
    for i in range(4):
        pltpu.async_copy(x_hbm.at[g0 + i], slabs[i], gsems[i])

    def outer(it, _):
        base = it * 8
        for j in range(8):
            p = j & 3
            sdyn = base + j
            pltpu.make_async_copy(x_hbm.at[g0], slabs[p], gsems[p]).wait()
            op = (j // 4) & 1
            if j % 4 == 0:
                @pl.when(it > 0)
                def _():
                    pltpu.make_async_copy(
                        outs[op], out_hbm.at[b, qbase], osems[op]).wait()
            _compute_slab(slabs[p], tabv, outs[op], j % 4)
            @pl.when(sdyn + 4 < _SLABS)
            def _():
                pltpu.async_copy(
                    x_hbm.at[g0 + sdyn + 4], slabs[p], gsems[p])
            if j % 4 == 3:
                pltpu.async_copy(
                    outs[op], out_hbm.at[b, qbase + it * 2 + op],
                    osems[op])
        return 0

    lax.fori_loop(0, _SLABS // 8, outer, 0)
    pltpu.make_async_copy(outs[0], out_hbm.at[b, qbase], osems[0]).wait()
    pltpu.make_async_copy(outs[1], out_hbm.at[b, qbase], osems[1]).wait()


_sc_kernel = functools.partial(
    pl.kernel,
    out_type=jax.ShapeDtypeStruct((_B, 32, 128, _C), jnp.float32),
    mesh=plsc.VectorSubcoreMesh(core_axis_name="c", subcore_axis_name="s"),
    scratch_types=[
        pltpu.VMEM((32, 16), jnp.int32),
        pltpu.VMEM((128, 128), jnp.float32),
        pltpu.VMEM((128, 128), jnp.float32),
        pltpu.VMEM((128, 128), jnp.float32),
        pltpu.VMEM((128, 128), jnp.float32),
        pltpu.VMEM((128, _C), jnp.float32),
        pltpu.VMEM((128, _C), jnp.float32),
        pltpu.SemaphoreType.DMA,
        pltpu.SemaphoreType.DMA,
        pltpu.SemaphoreType.DMA,
        pltpu.SemaphoreType.DMA,
        pltpu.SemaphoreType.DMA,
        pltpu.SemaphoreType.DMA,
    ],
    compiler_params=pltpu.CompilerParams(needs_layout_passes=False),
)(_sc_body)


def kernel(x, rp_map_idx):
    t = jnp.transpose(x, (0, 2, 3, 1))
    t = t.reshape(_B, _H, 8, 8, 4, 128)
    t = jnp.transpose(t, (0, 1, 2, 4, 3, 5))
    x3 = t.reshape(2 * _B * _H, 128, 128)

    rp = rp_map_idx.astype(jnp.int32)
    rowoff = (rp // 128) * 8
    coloff = rp % 128
    rtab = jnp.transpose(rowoff.reshape(4, 16, _K), (0, 2, 1)).reshape(16, 16)
    ctab = jnp.transpose(coloff.reshape(4, 16, _K), (0, 2, 1)).reshape(16, 16)
    tab = jnp.concatenate([rtab, ctab], axis=0)

    y6 = _sc_kernel(x3, tab)
    y = y6.reshape(_B, 32, 2, _W, _C)
    y = jnp.transpose(y, (0, 4, 1, 2, 3))
    return y.reshape(_B, _C, _H, _W)


# speedup vs baseline: 1.2060x; 1.1613x over previous
import functools

import jax
import jax.numpy as jnp
from jax import lax
from jax.experimental import pallas as pl
from jax.experimental.pallas import tpu as pltpu
from jax.experimental.pallas import tpu_sc as plsc

_B, _TB, _H, _W = 16, 512, 64, 64
_C, _K = 64, 4
_NW = 32
_SLABS = 64
_THRESH = 2.0


def _compute_slab(slab, tabv, out_v, quarter):
    rtabs = [tabv[r, :] for r in range(16)]
    ctabs = [tabv[16 + r, :] for r in range(16)]

    @plsc.parallel_loop(0, 32, unroll=4)
    def pbody(p):
        rowb = (p >> 3) * 32 + (p & 7)
        for j in range(4):
            acc = None
            for k in range(_K):
                ir = rtabs[4 * j + k] + rowb
                g = plsc.load_gather(slab, [ir, ctabs[4 * j + k]])
                acc = g if k == 0 else acc + g
            y = jnp.where(acc >= _THRESH, jnp.float32(1.0), jnp.float32(0.0))
            out_v[quarter * 32 + p, pl.ds(16 * j, 16)] = y


def _sc_body(x_hbm, tab_hbm, out_hbm,
             tabv, slab_a, slab_b, slab_c, slab_d, ov_a, ov_b,
             gs_a, gs_b, gs_c, gs_d, os_a, os_b):
    wid = lax.axis_index("s") * 2 + lax.axis_index("c")
    b = wid // 2
    qbase = (wid % 2) * 16
    pltpu.sync_copy(tab_hbm, tabv)

    slabs = [slab_a, slab_b, slab_c, slab_d]
    outs = [ov_a, ov_b]
    gsems = [gs_a, gs_b, gs_c, gs_d]
    osems = [os_a, os_b]

    g0 = wid * _SLABS
    for i in range(4):
        pltpu.async_copy(x_hbm.at[g0 + i], slabs[i], gsems[i])

    def outer(it, _):
        base = it * 8
        for j in range(8):
            p = j & 3
            sdyn = base + j
            pltpu.make_async_copy(x_hbm.at[g0], slabs[p], gsems[p]).wait()
            op = (j // 4) & 1
            if j % 4 == 0:
                @pl.when(it > 0)
                def _():
                    pltpu.make_async_copy(
                        outs[op], out_hbm.at[b, qbase], osems[op]).wait()
            _compute_slab(slabs[p], tabv, outs[op], j % 4)
            @pl.when(sdyn + 4 < _SLABS)
            def _():
                pltpu.async_copy(
                    x_hbm.at[g0 + sdyn + 4], slabs[p], gsems[p])
            if j % 4 == 3:
                pltpu.async_copy(
                    outs[op], out_hbm.at[b, qbase + it * 2 + op],
                    osems[op])
        return 0

    lax.fori_loop(0, _SLABS // 8, outer, 0)
    pltpu.make_async_copy(outs[0], out_hbm.at[b, qbase], osems[0]).wait()
    pltpu.make_async_copy(outs[1], out_hbm.at[b, qbase], osems[1]).wait()


_sc_kernel = functools.partial(
    pl.kernel,
    out_type=jax.ShapeDtypeStruct((_B, 32, 128, _C), jnp.float32),
    mesh=plsc.VectorSubcoreMesh(core_axis_name="c", subcore_axis_name="s"),
    scratch_types=[
        pltpu.VMEM((32, 16), jnp.int32),
        pltpu.VMEM((128, 128), jnp.float32),
        pltpu.VMEM((128, 128), jnp.float32),
        pltpu.VMEM((128, 128), jnp.float32),
        pltpu.VMEM((128, 128), jnp.float32),
        pltpu.VMEM((128, _C), jnp.float32),
        pltpu.VMEM((128, _C), jnp.float32),
        pltpu.SemaphoreType.DMA,
        pltpu.SemaphoreType.DMA,
        pltpu.SemaphoreType.DMA,
        pltpu.SemaphoreType.DMA,
        pltpu.SemaphoreType.DMA,
        pltpu.SemaphoreType.DMA,
    ],
    compiler_params=pltpu.CompilerParams(needs_layout_passes=False),
)(_sc_body)


def kernel(x, rp_map_idx):
    t = jnp.transpose(x, (0, 2, 3, 1))
    t = t.reshape(_B, _H, 8, 8, 4, 128)
    t = jnp.transpose(t, (0, 1, 2, 4, 3, 5))
    x3 = t.reshape(2 * _B * _H, 128, 128)

    rp = rp_map_idx.astype(jnp.int32)
    rowoff = (rp // 128) * 8
    coloff = rp % 128
    rtab = jnp.transpose(rowoff.reshape(4, 16, _K), (0, 2, 1)).reshape(16, 16)
    ctab = jnp.transpose(coloff.reshape(4, 16, _K), (0, 2, 1)).reshape(16, 16)
    tab = jnp.concatenate([rtab, ctab], axis=0)

    y6 = _sc_kernel(x3, tab)
    y = y6.reshape(_B, 32, 2, _W, _C)
    y = jnp.transpose(y, (0, 4, 1, 2, 3))
    return y.reshape(_B, _C, _H, _W)
